# Initial kernel scaffold; baseline (speedup 1.0000x reference)
import jax
import jax.numpy as jnp
from jax.experimental import pallas as pl


def kernel(inputs, W1, b1, W2, b2, W3, b3):
    raise NotImplementedError("write your pallas kernel here")


# speedup vs baseline: 1.5312x; 1.5312x over previous
import jax
import jax.numpy as jnp
from jax.experimental import pallas as pl

H = 128
W = 128
CIN = 256
CMID = 512
NSC = 6
NBB = 12
BH = 8
M = BH * W


def _fused_rpn_body(x_ref, w1_ref, b1_ref, w23_ref, b23_ref, score_ref, bbox_ref):
    i = pl.program_id(0)
    acc = jnp.zeros((M, CMID), jnp.float32)
    for kh in range(3):
        rows = x_ref[pl.ds(i * BH + kh, BH)]
        for kw in range(3):
            op = rows[:, kw:kw + W, :].reshape(M, CIN)
            acc += jnp.dot(op, w1_ref[kh, kw],
                           preferred_element_type=jnp.float32)
    y = jnp.maximum(acc + b1_ref[...], 0.0).astype(jnp.bfloat16)
    heads = jnp.dot(y, w23_ref[...],
                    preferred_element_type=jnp.float32) + b23_ref[...]
    score_ref[...] = heads[:, :NSC]
    bbox_ref[...] = heads[:, NSC:NSC + NBB]


def kernel(inputs, W1, b1, W2, b2, W3, b3):
    x = jnp.pad(inputs[0].astype(jnp.bfloat16), ((1, 1), (1, 1), (0, 0)))
    w1 = W1.astype(jnp.bfloat16)
    w23 = jnp.concatenate([W2[0, 0], W3[0, 0]], axis=1).astype(jnp.bfloat16)
    b23 = jnp.concatenate([b2, b3]).reshape(1, NSC + NBB)

    score, bbox = pl.pallas_call(
        _fused_rpn_body,
        grid=(H // BH,),
        in_specs=[
            pl.BlockSpec((H + 2, W + 2, CIN), lambda i: (0, 0, 0)),
            pl.BlockSpec((3, 3, CIN, CMID), lambda i: (0, 0, 0, 0)),
            pl.BlockSpec((1, CMID), lambda i: (0, 0)),
            pl.BlockSpec((CMID, NSC + NBB), lambda i: (0, 0)),
            pl.BlockSpec((1, NSC + NBB), lambda i: (0, 0)),
        ],
        out_specs=[
            pl.BlockSpec((M, NSC), lambda i: (i, 0)),
            pl.BlockSpec((M, NBB), lambda i: (i, 0)),
        ],
        out_shape=[
            jax.ShapeDtypeStruct((H * W, NSC), jnp.float32),
            jax.ShapeDtypeStruct((H * W, NBB), jnp.float32),
        ],
    )(x, w1, b1.reshape(1, CMID), w23, b23)

    return score, bbox.reshape(-1, 4)


# speedup vs baseline: 1.8188x; 1.1879x over previous
import jax
import jax.numpy as jnp
from jax.experimental import pallas as pl
from jax.experimental.pallas import tpu as pltpu

H = 128
W = 128
CIN = 256
CMID = 512
NSC = 6
NBB = 12
BH = 8
M = BH * W
NBLK = H // BH


def _fused_rpn_body(x_ref, w1_ref, w2_ref, w3_ref, b1_ref, b2_ref, b3_ref,
                    score_ref, bbox_ref, xp, w1s, w23s):
    i = pl.program_id(0)

    @pl.when(i == 0)
    def _prep():
        xp[1:H + 1, 1:W + 1, :] = x_ref[...].astype(jnp.bfloat16)
        zrow = jnp.zeros((1, W + 2, CIN), jnp.bfloat16)
        xp[0:1, :, :] = zrow
        xp[H + 1:H + 2, :, :] = zrow
        zcol = jnp.zeros((H + 2, 1, CIN), jnp.bfloat16)
        xp[:, 0:1, :] = zcol
        xp[:, W + 1:W + 2, :] = zcol
        w1s[...] = w1_ref[...].astype(jnp.bfloat16)
        w23s[:, :NSC] = w2_ref[...].astype(jnp.bfloat16)
        w23s[:, NSC:NSC + NBB] = w3_ref[...].astype(jnp.bfloat16)

    @pl.when(i > 0)
    def _block():
        j = i - 1
        acc = jnp.zeros((M, CMID), jnp.float32)
        for kh in range(3):
            rows = xp[pl.ds(j * BH + kh, BH)]
            for kw in range(3):
                op = rows[:, kw:kw + W, :].reshape(M, CIN)
                acc = acc + jnp.dot(op, w1s[kh, kw],
                                    preferred_element_type=jnp.float32)
        y = jnp.maximum(acc + b1_ref[...], 0.0).astype(jnp.bfloat16)
        heads = jnp.dot(y, w23s[...], preferred_element_type=jnp.float32)
        score_ref[...] = heads[:, :NSC] + b2_ref[...]
        bbox_ref[...] = heads[:, NSC:NSC + NBB] + b3_ref[...]


def kernel(inputs, W1, b1, W2, b2, W3, b3):
    score, bbox = pl.pallas_call(
        _fused_rpn_body,
        grid=(NBLK + 1,),
        in_specs=[
            pl.BlockSpec((H, W, CIN), lambda i: (0, 0, 0)),
            pl.BlockSpec((3, 3, CIN, CMID), lambda i: (0, 0, 0, 0)),
            pl.BlockSpec((CMID, NSC), lambda i: (0, 0)),
            pl.BlockSpec((CMID, NBB), lambda i: (0, 0)),
            pl.BlockSpec((1, CMID), lambda i: (0, 0)),
            pl.BlockSpec((1, NSC), lambda i: (0, 0)),
            pl.BlockSpec((1, NBB), lambda i: (0, 0)),
        ],
        out_specs=[
            pl.BlockSpec((M, NSC), lambda i: (jnp.maximum(i - 1, 0), 0)),
            pl.BlockSpec((M, NBB), lambda i: (jnp.maximum(i - 1, 0), 0)),
        ],
        out_shape=[
            jax.ShapeDtypeStruct((H * W, NSC), jnp.float32),
            jax.ShapeDtypeStruct((H * W, NBB), jnp.float32),
        ],
        scratch_shapes=[
            pltpu.VMEM((H + 2, W + 2, CIN), jnp.bfloat16),
            pltpu.VMEM((3, 3, CIN, CMID), jnp.bfloat16),
            pltpu.VMEM((CMID, NSC + NBB), jnp.bfloat16),
        ],
    )(inputs[0], W1, W2[0, 0], W3[0, 0],
      b1.reshape(1, CMID), b2.reshape(1, NSC), b3.reshape(1, NBB))

    return score, bbox.reshape(-1, 4)


# speedup vs baseline: 2.1735x; 1.1950x over previous
import jax
import jax.numpy as jnp
from jax.experimental import pallas as pl
from jax.experimental.pallas import tpu as pltpu

H = 128
W = 128
CIN = 256
CMID = 512
NSC = 6
NBB = 12
BH = 8
M = BH * W
NBLK = H // BH


def _fused_rpn_body(x_ref, w1_ref, w2_ref, w3_ref, b1_ref, b2_ref, b3_ref,
                    score_ref, bbox_ref, xp, w1s, w23s):
    i = pl.program_id(0)

    @pl.when(i == 0)
    def _prep():
        xp[1:H + 1, 1:W + 1, :] = x_ref[...].astype(jnp.bfloat16)
        zrow = jnp.zeros((1, W + 2, CIN), jnp.bfloat16)
        xp[0:1, :, :] = zrow
        xp[H + 1:H + 2, :, :] = zrow
        zcol = jnp.zeros((H + 2, 1, CIN), jnp.bfloat16)
        xp[:, 0:1, :] = zcol
        xp[:, W + 1:W + 2, :] = zcol
        w1s[...] = w1_ref[...].astype(jnp.bfloat16)
        w23s[:, :NSC] = w2_ref[...].astype(jnp.bfloat16)
        w23s[:, NSC:NSC + NBB] = w3_ref[...].astype(jnp.bfloat16)

    @pl.when(i > 0)
    def _block():
        j = i - 1
        acc = jnp.zeros((M, CMID), jnp.float32)
        for kh in range(3):
            rows = xp[pl.ds(j * BH + kh, BH)]
            for kw in range(3):
                op = rows[:, kw:kw + W, :].reshape(M, CIN)
                acc = acc + jnp.dot(op, w1s[kh, kw],
                                    preferred_element_type=jnp.float32)
        y = jnp.maximum(acc + b1_ref[...], 0.0).astype(jnp.bfloat16)
        heads = jnp.dot(y, w23s[...], preferred_element_type=jnp.float32)
        score_ref[...] = heads[:, :NSC] + b2_ref[...]
        for a in range(3):
            bbox_ref[pl.Slice(a, M, 3), :] = (
                heads[:, NSC + 4 * a:NSC + 4 * (a + 1)]
                + b3_ref[:, 4 * a:4 * (a + 1)])


def kernel(inputs, W1, b1, W2, b2, W3, b3):
    score, bbox = pl.pallas_call(
        _fused_rpn_body,
        grid=(NBLK + 1,),
        in_specs=[
            pl.BlockSpec((H, W, CIN), lambda i: (0, 0, 0)),
            pl.BlockSpec((3, 3, CIN, CMID), lambda i: (0, 0, 0, 0)),
            pl.BlockSpec((CMID, NSC), lambda i: (0, 0)),
            pl.BlockSpec((CMID, NBB), lambda i: (0, 0)),
            pl.BlockSpec((1, CMID), lambda i: (0, 0)),
            pl.BlockSpec((1, NSC), lambda i: (0, 0)),
            pl.BlockSpec((1, NBB), lambda i: (0, 0)),
        ],
        out_specs=[
            pl.BlockSpec((M, NSC), lambda i: (jnp.maximum(i - 1, 0), 0)),
            pl.BlockSpec((3 * M, 4), lambda i: (jnp.maximum(i - 1, 0), 0)),
        ],
        out_shape=[
            jax.ShapeDtypeStruct((H * W, NSC), jnp.float32),
            jax.ShapeDtypeStruct((3 * H * W, 4), jnp.float32),
        ],
        scratch_shapes=[
            pltpu.VMEM((H + 2, W + 2, CIN), jnp.bfloat16),
            pltpu.VMEM((3, 3, CIN, CMID), jnp.bfloat16),
            pltpu.VMEM((CMID, NSC + NBB), jnp.bfloat16),
        ],
    )(inputs[0], W1, W2[0, 0], W3[0, 0],
      b1.reshape(1, CMID), b2.reshape(1, NSC), b3.reshape(1, NBB))

    return score, bbox
